# baseline (speedup 1.0000x reference)
import functools

import jax
import jax.numpy as jnp
from jax import lax
from jax.experimental import pallas as pl
from jax.experimental.pallas import tpu as pltpu
from jax.experimental.pallas import tpu_sc as plsc

_B = 16384
_L = 50
_V1, _V2, _V3 = 101, 101, 49
_E1, _E2, _E3 = 96, 96, 64
_TE = _E1 + _E2 + _E3
_MD = 256

_NC, _NS = 2, 16
_NW = _NC * _NS
_VS = _V1 + _V2 + _V3
_TV = _L * _VS
_CPT = _MD // _NW
_QPT = _CPT // 2
_TFL = _QPT * _TV
_NJ = 150
_CB = 128
_NCH = _B // _CB
_UNROLL = 15


def _tables_body(w_ref, e1_ref, e2_ref, e3_ref, t_ref):
    w = w_ref[0]
    dn = (((1,), (1,)), ((), ()))
    t_ref[0, 0:_V1, :] = lax.dot_general(
        e1_ref[...], w[:, 0:_E1], dn,
        preferred_element_type=jnp.float32).astype(jnp.bfloat16)
    t_ref[0, _V1:_V1 + _V2, :] = lax.dot_general(
        e2_ref[...], w[:, _E1:_E1 + _E2], dn,
        preferred_element_type=jnp.float32).astype(jnp.bfloat16)
    t_ref[0, _V1 + _V2:_VS, :] = lax.dot_general(
        e3_ref[...], w[:, _E1 + _E2:_TE], dn,
        preferred_element_type=jnp.float32).astype(jnp.bfloat16)


def _make_tables(W1, emb1, emb2, emb3):
    w1r = W1.reshape(_MD, _L, _TE).transpose(1, 0, 2)
    t = pl.pallas_call(
        _tables_body,
        grid=(_L,),
        in_specs=[
            pl.BlockSpec((1, _MD, _TE), lambda l: (l, 0, 0)),
            pl.BlockSpec((_V1, _E1), lambda l: (0, 0)),
            pl.BlockSpec((_V2, _E2), lambda l: (0, 0)),
            pl.BlockSpec((_V3, _E3), lambda l: (0, 0)),
        ],
        out_specs=pl.BlockSpec((1, _VS, _MD), lambda l: (l, 0, 0)),
        out_shape=jax.ShapeDtypeStruct((_L, _VS, _MD), jnp.bfloat16),
    )(w1r, emb1, emb2, emb3)
    return t.reshape(_TV, _MD)


def _bag_body(ts_h, idx_h, out_h, tbl, ha, hb, oa, ob,
              sem_a, sem_b, sem_oa, sem_ob):
    cid = lax.axis_index("c")
    sid = lax.axis_index("s")
    wid = sid * _NC + cid

    pltpu.sync_copy(ts_h.at[pl.ds(wid * _TFL, _TFL)], tbl)

    def issue(ch, buf, sem):
        pltpu.async_copy(
            idx_h.at[pl.ds(0, _NJ), pl.ds(ch * _CB, _CB)], buf, sem)

    def drain(buf, sem):
        pltpu.make_async_copy(idx_h.at[pl.ds(0, _NJ), pl.ds(0, _CB)],
                              buf, sem).wait()

    def drain_out(ob, sem):
        pltpu.make_async_copy(ob, out_h.at[0, :, pl.ds(0, _CB)], sem).wait()

    def accum_chunk(ch, hbuf, ob, sem):
        for bb in range(_CB // 16):
            def jbody(j8, a, bb=bb):
                jb = j8 * _UNROLL
                for jj in range(_UNROLL):
                    iv = hbuf[jb + jj, pl.ds(bb * 16, 16)]
                    for q in range(_QPT):
                        g = plsc.load_gather(tbl, [iv + (q * _TV)])
                        ab = plsc.bitcast(g, jnp.bfloat16)
                        lo, hi = plsc.unpack(
                            ab, format=plsc.PackFormat.INTERLEAVED)
                        a = (a[:2 * q]
                             + (a[2 * q] + lo, a[2 * q + 1] + hi)
                             + a[2 * q + 2:])
                return a

            acc = plsc.parallel_loop(
                0, _NJ // _UNROLL,
                carry=(jnp.zeros((16,), jnp.float32),) * _CPT)(jbody)
            for c in range(_CPT):
                ob[c, pl.ds(bb * 16, 16)] = acc[c]
        pltpu.async_copy(ob, out_h.at[wid, :, pl.ds(ch * _CB, _CB)], sem)

    issue(0, ha, sem_a)

    def pair_body(k, carry):
        ch = k * 2
        issue(ch + 1, hb, sem_b)
        drain(ha, sem_a)

        @pl.when(k > 0)
        def _():
            drain_out(oa, sem_oa)

        accum_chunk(ch, ha, oa, sem_oa)

        @pl.when(k < _NCH // 2 - 1)
        def _():
            issue(ch + 2, ha, sem_a)

        drain(hb, sem_b)

        @pl.when(k > 0)
        def _():
            drain_out(ob, sem_ob)

        accum_chunk(ch + 1, hb, ob, sem_ob)
        return carry

    lax.fori_loop(0, _NCH // 2, pair_body, 0)
    drain_out(oa, sem_oa)
    drain_out(ob, sem_ob)


def _bag(ts, idxt):
    mesh = plsc.VectorSubcoreMesh(core_axis_name="c", subcore_axis_name="s",
                                  num_cores=_NC, num_subcores=_NS)
    return pl.kernel(
        _bag_body,
        out_type=jax.ShapeDtypeStruct((_NW, _CPT, _B), jnp.float32),
        mesh=mesh,
        compiler_params=pltpu.CompilerParams(needs_layout_passes=False),
        scratch_types=[
            pltpu.VMEM((_TFL,), jnp.int32),
            pltpu.VMEM((_NJ, _CB), jnp.int32),
            pltpu.VMEM((_NJ, _CB), jnp.int32),
            pltpu.VMEM((_CPT, _CB), jnp.float32),
            pltpu.VMEM((_CPT, _CB), jnp.float32),
            pltpu.SemaphoreType.DMA,
            pltpu.SemaphoreType.DMA,
            pltpu.SemaphoreType.DMA,
            pltpu.SemaphoreType.DMA,
        ],
    )(ts, idxt)


_MLP_BLK = 1024


def _mlp_body(h_ref, b1_ref, w2_ref, b2_ref, w3_ref, b3_ref, o_ref):
    xt = h_ref[...].reshape(_MD, _MLP_BLK)
    xt = jnp.maximum(xt + b1_ref[...], 0.0)
    dn = (((1,), (0,)), ((), ()))
    h2 = lax.dot_general(w2_ref[...], xt, dn,
                         preferred_element_type=jnp.float32) + b2_ref[...]
    h2 = jnp.maximum(h2, 0.0)
    o = lax.dot_general(w3_ref[...], h2, dn,
                        preferred_element_type=jnp.float32) + b3_ref[0, 0]
    o_ref[...] = o[0:1, :]


def _mlp(strips, b1, W2, b2, W3, b3):
    b1b = jnp.broadcast_to(b1[:, None], (_MD, _MLP_BLK))
    b2b = jnp.broadcast_to(b2[:, None], (_MD, _MLP_BLK))
    out = pl.pallas_call(
        _mlp_body,
        grid=(_B // _MLP_BLK,),
        in_specs=[
            pl.BlockSpec((_NW, _CPT, _MLP_BLK), lambda i: (0, 0, i)),
            pl.BlockSpec((_MD, _MLP_BLK), lambda i: (0, 0)),
            pl.BlockSpec((_MD, _MD), lambda i: (0, 0)),
            pl.BlockSpec((_MD, _MLP_BLK), lambda i: (0, 0)),
            pl.BlockSpec((8, _MD), lambda i: (0, 0)),
            pl.BlockSpec((1, 1), lambda i: (0, 0)),
        ],
        out_specs=pl.BlockSpec((1, _MLP_BLK), lambda i: (0, i)),
        out_shape=jax.ShapeDtypeStruct((1, _B), jnp.float32),
    )(strips, b1b, W2, b2b,
      jnp.pad(W3, ((0, 7), (0, 0))), b3.reshape(1, 1))
    return out.reshape(_B, 1)


def kernel(x1, x2, x3, mask, device, emb1, emb2, emb3,
           W1, b1, W2, b2, W3, b3):
    del mask, device
    tflat = _make_tables(W1, emb1, emb2, emb3)
    ts = lax.bitcast_convert_type(
        tflat.reshape(_TV, _MD // 2, 2).transpose(1, 0, 2),
        jnp.int32).reshape(_NW * _TFL)

    x1i, x2i, x3i = (x.astype(jnp.int32) for x in (x1, x2, x3))
    pos = jnp.arange(_L, dtype=jnp.int32)[None, :] * _VS
    idx = jnp.concatenate([
        x1i + pos,
        x2i + pos + _V1,
        x3i + pos + _V1 + _V2,
    ], axis=1).T

    strips = _bag(ts, idx)
    return _mlp(strips, b1, W2, b2, W3, b3)


# speedup vs baseline: 1.5793x; 1.5793x over previous
import functools

import jax
import jax.numpy as jnp
from jax import lax
from jax.experimental import pallas as pl
from jax.experimental.pallas import tpu as pltpu
from jax.experimental.pallas import tpu_sc as plsc

_B = 16384
_L = 50
_V1, _V2, _V3 = 101, 101, 49
_E1, _E2, _E3 = 96, 96, 64
_TE = _E1 + _E2 + _E3
_MD = 256

_NC, _NS = 2, 16
_NW = _NC * _NS
_VS = _V1 + _V2 + _V3
_TV = _L * _VS
_CPT = _MD // _NW
_QPT = _CPT // 2
_TFL = _QPT * _TV
_NJ = 150
_CB = 128
_NCH = _B // _CB
_UNROLL = 6


def _tables_body(w_ref, e1_ref, e2_ref, e3_ref, t_ref):
    w = w_ref[0]
    dn = (((1,), (1,)), ((), ()))
    t_ref[0, 0:_V1, :] = lax.dot_general(
        e1_ref[...], w[:, 0:_E1], dn,
        preferred_element_type=jnp.float32).astype(jnp.bfloat16)
    t_ref[0, _V1:_V1 + _V2, :] = lax.dot_general(
        e2_ref[...], w[:, _E1:_E1 + _E2], dn,
        preferred_element_type=jnp.float32).astype(jnp.bfloat16)
    t_ref[0, _V1 + _V2:_VS, :] = lax.dot_general(
        e3_ref[...], w[:, _E1 + _E2:_TE], dn,
        preferred_element_type=jnp.float32).astype(jnp.bfloat16)


def _make_tables(W1, emb1, emb2, emb3):
    w1r = W1.reshape(_MD, _L, _TE).transpose(1, 0, 2)
    t = pl.pallas_call(
        _tables_body,
        grid=(_L,),
        in_specs=[
            pl.BlockSpec((1, _MD, _TE), lambda l: (l, 0, 0)),
            pl.BlockSpec((_V1, _E1), lambda l: (0, 0)),
            pl.BlockSpec((_V2, _E2), lambda l: (0, 0)),
            pl.BlockSpec((_V3, _E3), lambda l: (0, 0)),
        ],
        out_specs=pl.BlockSpec((1, _VS, _MD), lambda l: (l, 0, 0)),
        out_shape=jax.ShapeDtypeStruct((_L, _VS, _MD), jnp.bfloat16),
    )(w1r, emb1, emb2, emb3)
    return t.reshape(_TV, _MD)


def _bag_body(ts_h, idx_h, out_h, tbl, ha, hb, oa, ob,
              sem_a, sem_b, sem_oa, sem_ob):
    cid = lax.axis_index("c")
    sid = lax.axis_index("s")
    wid = sid * _NC + cid

    pltpu.sync_copy(ts_h.at[pl.ds(wid * _TFL, _TFL)], tbl)

    def issue(ch, buf, sem):
        pltpu.async_copy(
            idx_h.at[pl.ds(0, _NJ), pl.ds(ch * _CB, _CB)], buf, sem)

    def drain(buf, sem):
        pltpu.make_async_copy(idx_h.at[pl.ds(0, _NJ), pl.ds(0, _CB)],
                              buf, sem).wait()

    def drain_out(ob, sem):
        pltpu.make_async_copy(ob, out_h.at[0, :, pl.ds(0, _CB)], sem).wait()

    def accum_chunk(ch, hbuf, ob, sem):
        for bb in range(_CB // 16):
            def jbody(j8, a, bb=bb):
                jb = j8 * _UNROLL
                for jj in range(_UNROLL):
                    iv = hbuf[jb + jj, pl.ds(bb * 16, 16)]
                    for q in range(_QPT):
                        g = plsc.load_gather(tbl, [iv + (q * _TV)])
                        ab = plsc.bitcast(g, jnp.bfloat16)
                        lo, hi = plsc.unpack(
                            ab, format=plsc.PackFormat.INTERLEAVED)
                        a = (a[:2 * q]
                             + (a[2 * q] + lo, a[2 * q + 1] + hi)
                             + a[2 * q + 2:])
                return a

            acc = plsc.parallel_loop(
                0, _NJ // _UNROLL,
                carry=(jnp.zeros((16,), jnp.float32),) * _CPT)(jbody)
            for c in range(_CPT):
                ob[c, pl.ds(bb * 16, 16)] = acc[c]
        pltpu.async_copy(ob, out_h.at[wid, :, pl.ds(ch * _CB, _CB)], sem)

    issue(0, ha, sem_a)

    def pair_body(k, carry):
        ch = k * 2
        issue(ch + 1, hb, sem_b)
        drain(ha, sem_a)

        @pl.when(k > 0)
        def _():
            drain_out(oa, sem_oa)

        accum_chunk(ch, ha, oa, sem_oa)

        @pl.when(k < _NCH // 2 - 1)
        def _():
            issue(ch + 2, ha, sem_a)

        drain(hb, sem_b)

        @pl.when(k > 0)
        def _():
            drain_out(ob, sem_ob)

        accum_chunk(ch + 1, hb, ob, sem_ob)
        return carry

    lax.fori_loop(0, _NCH // 2, pair_body, 0)
    drain_out(oa, sem_oa)
    drain_out(ob, sem_ob)


def _bag(ts, idxt):
    mesh = plsc.VectorSubcoreMesh(core_axis_name="c", subcore_axis_name="s",
                                  num_cores=_NC, num_subcores=_NS)
    return pl.kernel(
        _bag_body,
        out_type=jax.ShapeDtypeStruct((_NW, _CPT, _B), jnp.float32),
        mesh=mesh,
        compiler_params=pltpu.CompilerParams(needs_layout_passes=False),
        scratch_types=[
            pltpu.VMEM((_TFL,), jnp.int32),
            pltpu.VMEM((_NJ, _CB), jnp.int32),
            pltpu.VMEM((_NJ, _CB), jnp.int32),
            pltpu.VMEM((_CPT, _CB), jnp.float32),
            pltpu.VMEM((_CPT, _CB), jnp.float32),
            pltpu.SemaphoreType.DMA,
            pltpu.SemaphoreType.DMA,
            pltpu.SemaphoreType.DMA,
            pltpu.SemaphoreType.DMA,
        ],
    )(ts, idxt)


_MLP_BLK = 1024


def _mlp_body(h_ref, b1_ref, w2_ref, b2_ref, w3_ref, b3_ref, o_ref):
    xt = h_ref[...].reshape(_MD, _MLP_BLK)
    xt = jnp.maximum(xt + b1_ref[...], 0.0)
    dn = (((1,), (0,)), ((), ()))
    h2 = lax.dot_general(w2_ref[...], xt, dn,
                         preferred_element_type=jnp.float32) + b2_ref[...]
    h2 = jnp.maximum(h2, 0.0)
    o = lax.dot_general(w3_ref[...], h2, dn,
                        preferred_element_type=jnp.float32) + b3_ref[0, 0]
    o_ref[...] = o[0:1, :]


def _mlp(strips, b1, W2, b2, W3, b3):
    b1b = jnp.broadcast_to(b1[:, None], (_MD, _MLP_BLK))
    b2b = jnp.broadcast_to(b2[:, None], (_MD, _MLP_BLK))
    out = pl.pallas_call(
        _mlp_body,
        grid=(_B // _MLP_BLK,),
        in_specs=[
            pl.BlockSpec((_NW, _CPT, _MLP_BLK), lambda i: (0, 0, i)),
            pl.BlockSpec((_MD, _MLP_BLK), lambda i: (0, 0)),
            pl.BlockSpec((_MD, _MD), lambda i: (0, 0)),
            pl.BlockSpec((_MD, _MLP_BLK), lambda i: (0, 0)),
            pl.BlockSpec((8, _MD), lambda i: (0, 0)),
            pl.BlockSpec((1, 1), lambda i: (0, 0)),
        ],
        out_specs=pl.BlockSpec((1, _MLP_BLK), lambda i: (0, i)),
        out_shape=jax.ShapeDtypeStruct((1, _B), jnp.float32),
    )(strips, b1b, W2, b2b,
      jnp.pad(W3, ((0, 7), (0, 0))), b3.reshape(1, 1))
    return out.reshape(_B, 1)


def kernel(x1, x2, x3, mask, device, emb1, emb2, emb3,
           W1, b1, W2, b2, W3, b3):
    del mask, device
    tflat = _make_tables(W1, emb1, emb2, emb3)
    ts = lax.bitcast_convert_type(
        tflat.reshape(_TV, _MD // 2, 2).transpose(1, 0, 2),
        jnp.int32).reshape(_NW * _TFL)

    x1i, x2i, x3i = (x.astype(jnp.int32) for x in (x1, x2, x3))
    pos = jnp.arange(_L, dtype=jnp.int32)[None, :] * _VS
    idx = jnp.concatenate([
        x1i + pos,
        x2i + pos + _V1,
        x3i + pos + _V1 + _V2,
    ], axis=1).T

    strips = _bag(ts, idx)
    return _mlp(strips, b1, W2, b2, W3, b3)


# speedup vs baseline: 1.6033x; 1.0152x over previous
import functools

import jax
import jax.numpy as jnp
from jax import lax
from jax.experimental import pallas as pl
from jax.experimental.pallas import tpu as pltpu
from jax.experimental.pallas import tpu_sc as plsc

_B = 16384
_L = 50
_V1, _V2, _V3 = 101, 101, 49
_E1, _E2, _E3 = 96, 96, 64
_TE = _E1 + _E2 + _E3
_MD = 256

_NC, _NS = 2, 16
_NW = _NC * _NS
_VS = _V1 + _V2 + _V3
_TV = _L * _VS
_CPT = _MD // _NW
_QPT = _CPT // 2
_TFL = _QPT * _TV
_NJ = 150
_CB = 128
_NCH = _B // _CB
_UNROLL = 3


def _tables_body(w_ref, e1_ref, e2_ref, e3_ref, t_ref):
    w = w_ref[0]
    dn = (((1,), (1,)), ((), ()))
    t_ref[0, 0:_V1, :] = lax.dot_general(
        e1_ref[...], w[:, 0:_E1], dn,
        preferred_element_type=jnp.float32).astype(jnp.bfloat16)
    t_ref[0, _V1:_V1 + _V2, :] = lax.dot_general(
        e2_ref[...], w[:, _E1:_E1 + _E2], dn,
        preferred_element_type=jnp.float32).astype(jnp.bfloat16)
    t_ref[0, _V1 + _V2:_VS, :] = lax.dot_general(
        e3_ref[...], w[:, _E1 + _E2:_TE], dn,
        preferred_element_type=jnp.float32).astype(jnp.bfloat16)


def _make_tables(W1, emb1, emb2, emb3):
    w1r = W1.reshape(_MD, _L, _TE).transpose(1, 0, 2)
    t = pl.pallas_call(
        _tables_body,
        grid=(_L,),
        in_specs=[
            pl.BlockSpec((1, _MD, _TE), lambda l: (l, 0, 0)),
            pl.BlockSpec((_V1, _E1), lambda l: (0, 0)),
            pl.BlockSpec((_V2, _E2), lambda l: (0, 0)),
            pl.BlockSpec((_V3, _E3), lambda l: (0, 0)),
        ],
        out_specs=pl.BlockSpec((1, _VS, _MD), lambda l: (l, 0, 0)),
        out_shape=jax.ShapeDtypeStruct((_L, _VS, _MD), jnp.bfloat16),
    )(w1r, emb1, emb2, emb3)
    return t.reshape(_TV, _MD)


def _bag_body(ts_h, idx_h, out_h, tbl, ha, hb, oa, ob,
              sem_a, sem_b, sem_oa, sem_ob):
    cid = lax.axis_index("c")
    sid = lax.axis_index("s")
    wid = sid * _NC + cid

    pltpu.sync_copy(ts_h.at[pl.ds(wid * _TFL, _TFL)], tbl)

    def issue(ch, buf, sem):
        pltpu.async_copy(
            idx_h.at[pl.ds(0, _NJ), pl.ds(ch * _CB, _CB)], buf, sem)

    def drain(buf, sem):
        pltpu.make_async_copy(idx_h.at[pl.ds(0, _NJ), pl.ds(0, _CB)],
                              buf, sem).wait()

    def drain_out(ob, sem):
        pltpu.make_async_copy(ob, out_h.at[0, :, pl.ds(0, _CB)], sem).wait()

    def accum_chunk(ch, hbuf, ob, sem):
        for bb in range(_CB // 16):
            def jbody(j8, a, bb=bb):
                jb = j8 * _UNROLL
                for jj in range(_UNROLL):
                    iv = hbuf[jb + jj, pl.ds(bb * 16, 16)]
                    for q in range(_QPT):
                        g = plsc.load_gather(tbl, [iv + (q * _TV)])
                        ab = plsc.bitcast(g, jnp.bfloat16)
                        lo, hi = plsc.unpack(
                            ab, format=plsc.PackFormat.INTERLEAVED)
                        a = (a[:2 * q]
                             + (a[2 * q] + lo, a[2 * q + 1] + hi)
                             + a[2 * q + 2:])
                return a

            acc = plsc.parallel_loop(
                0, _NJ // _UNROLL,
                carry=(jnp.zeros((16,), jnp.float32),) * _CPT)(jbody)
            for c in range(_CPT):
                ob[c, pl.ds(bb * 16, 16)] = acc[c]
        pltpu.async_copy(ob, out_h.at[wid, :, pl.ds(ch * _CB, _CB)], sem)

    issue(0, ha, sem_a)

    def pair_body(k, carry):
        ch = k * 2
        issue(ch + 1, hb, sem_b)
        drain(ha, sem_a)

        @pl.when(k > 0)
        def _():
            drain_out(oa, sem_oa)

        accum_chunk(ch, ha, oa, sem_oa)

        @pl.when(k < _NCH // 2 - 1)
        def _():
            issue(ch + 2, ha, sem_a)

        drain(hb, sem_b)

        @pl.when(k > 0)
        def _():
            drain_out(ob, sem_ob)

        accum_chunk(ch + 1, hb, ob, sem_ob)
        return carry

    lax.fori_loop(0, _NCH // 2, pair_body, 0)
    drain_out(oa, sem_oa)
    drain_out(ob, sem_ob)


def _bag(ts, idxt):
    mesh = plsc.VectorSubcoreMesh(core_axis_name="c", subcore_axis_name="s",
                                  num_cores=_NC, num_subcores=_NS)
    return pl.kernel(
        _bag_body,
        out_type=jax.ShapeDtypeStruct((_NW, _CPT, _B), jnp.float32),
        mesh=mesh,
        compiler_params=pltpu.CompilerParams(needs_layout_passes=False),
        scratch_types=[
            pltpu.VMEM((_TFL,), jnp.int32),
            pltpu.VMEM((_NJ, _CB), jnp.int32),
            pltpu.VMEM((_NJ, _CB), jnp.int32),
            pltpu.VMEM((_CPT, _CB), jnp.float32),
            pltpu.VMEM((_CPT, _CB), jnp.float32),
            pltpu.SemaphoreType.DMA,
            pltpu.SemaphoreType.DMA,
            pltpu.SemaphoreType.DMA,
            pltpu.SemaphoreType.DMA,
        ],
    )(ts, idxt)


_MLP_BLK = 1024


def _mlp_body(h_ref, b1_ref, w2_ref, b2_ref, w3_ref, b3_ref, o_ref):
    xt = h_ref[...].reshape(_MD, _MLP_BLK)
    xt = jnp.maximum(xt + b1_ref[...], 0.0)
    dn = (((1,), (0,)), ((), ()))
    h2 = lax.dot_general(w2_ref[...], xt, dn,
                         preferred_element_type=jnp.float32) + b2_ref[...]
    h2 = jnp.maximum(h2, 0.0)
    o = lax.dot_general(w3_ref[...], h2, dn,
                        preferred_element_type=jnp.float32) + b3_ref[0, 0]
    o_ref[...] = o[0:1, :]


def _mlp(strips, b1, W2, b2, W3, b3):
    b1b = jnp.broadcast_to(b1[:, None], (_MD, _MLP_BLK))
    b2b = jnp.broadcast_to(b2[:, None], (_MD, _MLP_BLK))
    out = pl.pallas_call(
        _mlp_body,
        grid=(_B // _MLP_BLK,),
        in_specs=[
            pl.BlockSpec((_NW, _CPT, _MLP_BLK), lambda i: (0, 0, i)),
            pl.BlockSpec((_MD, _MLP_BLK), lambda i: (0, 0)),
            pl.BlockSpec((_MD, _MD), lambda i: (0, 0)),
            pl.BlockSpec((_MD, _MLP_BLK), lambda i: (0, 0)),
            pl.BlockSpec((8, _MD), lambda i: (0, 0)),
            pl.BlockSpec((1, 1), lambda i: (0, 0)),
        ],
        out_specs=pl.BlockSpec((1, _MLP_BLK), lambda i: (0, i)),
        out_shape=jax.ShapeDtypeStruct((1, _B), jnp.float32),
    )(strips, b1b, W2, b2b,
      jnp.pad(W3, ((0, 7), (0, 0))), b3.reshape(1, 1))
    return out.reshape(_B, 1)


def kernel(x1, x2, x3, mask, device, emb1, emb2, emb3,
           W1, b1, W2, b2, W3, b3):
    del mask, device
    tflat = _make_tables(W1, emb1, emb2, emb3)
    ts = lax.bitcast_convert_type(
        tflat.reshape(_TV, _MD // 2, 2).transpose(1, 0, 2),
        jnp.int32).reshape(_NW * _TFL)

    x1i, x2i, x3i = (x.astype(jnp.int32) for x in (x1, x2, x3))
    pos = jnp.arange(_L, dtype=jnp.int32)[None, :] * _VS
    idx = jnp.concatenate([
        x1i + pos,
        x2i + pos + _V1,
        x3i + pos + _V1 + _V2,
    ], axis=1).T

    strips = _bag(ts, idx)
    return _mlp(strips, b1, W2, b2, W3, b3)
